# baseline (speedup 1.0000x reference)
import functools

import numpy as np
import jax
import jax.numpy as jnp
from jax import lax
from jax.experimental import pallas as pl
from jax.experimental.pallas import tpu as pltpu
from jax.experimental.pallas import tpu_sc as plsc

VOCAB = 100000
D_MODEL = 768
BATCH = 4
SEQ = 2048
SCALE = float(np.sqrt(float(D_MODEL)))

NC = 2
NS = 16
NW = NC * NS
PB = SEQ // NW
C = 64
NCH = PB // C
LG = D_MODEL // 16
LGP = D_MODEL // 32
NB = 2
NCHT = BATCH * NCH


def _positional_encoding_np(length, depth):
    half_depth = depth // 2
    positions = np.arange(length)[:, np.newaxis]
    exponents = np.arange(half_depth)[np.newaxis, :] * 2 / depth
    denom = 10000 ** exponents
    angles = positions / denom
    pos_encoding = np.zeros((length, depth), dtype=np.float64)
    pos_encoding[:, ::2] = np.sin(angles)
    pos_encoding[:, 1::2] = np.cos(angles)
    return pos_encoding.astype(np.float32)


def _pack_pos_bf16_np(pos):
    u = pos.view(np.uint32).astype(np.uint64)
    bf = ((u + 0x7FFF + ((u >> 16) & 1)) >> 16).astype(np.uint32)
    bf = bf.reshape(pos.shape[0], LGP, 2, 16)
    packed = (bf[:, :, 0, :] << 16) | bf[:, :, 1, :]
    return packed.reshape(pos.shape[0], D_MODEL // 2).view(np.int32)


_MESH = plsc.VectorSubcoreMesh(core_axis_name="c", subcore_axis_name="s")


@functools.partial(
    pl.kernel,
    mesh=_MESH,
    out_type=jax.ShapeDtypeStruct((BATCH * SEQ, D_MODEL), jnp.float32),
    scratch_types=[
        pltpu.VMEM((BATCH, NCH, C), jnp.int32),
        pltpu.VMEM((PB, D_MODEL // 2), jnp.int32),
    ] + [pltpu.VMEM((C, D_MODEL), jnp.float32) for _ in range(NB)]
      + [pltpu.SemaphoreType.DMA for _ in range(2 * NB)],
)
def _embed(table_hbm, idx_hbm, pos_hbm, out_hbm, idx_v, pos_v, *rest):
    bufs = rest[:NB]
    gsems = rest[NB:2 * NB]
    wsems = rest[2 * NB:]

    wid = lax.axis_index("s") * NC + lax.axis_index("c")
    pbase = wid * PB

    pltpu.sync_copy(idx_hbm.at[wid], idx_v)
    pltpu.sync_copy(pos_hbm.at[pl.ds(pbase, PB)], pos_v)

    def start_gather(k, bi):
        b, c = divmod(k, NCH)
        return pltpu.async_copy(table_hbm.at[idx_v.at[b, c]], bufs[bi],
                                gsems[bi])

    def start_write(k, bi):
        b, c = divmod(k, NCH)
        out_base = b * SEQ + pbase + c * C
        return pltpu.async_copy(bufs[bi], out_hbm.at[pl.ds(out_base, C)],
                                wsems[bi])

    gcopy = [None] * NB
    wcopy = [None] * NB
    for k in range(min(NB - 1, NCHT)):
        gcopy[k % NB] = start_gather(k, k % NB)

    hi_mask = jnp.int32(-65536)

    for k in range(NCHT):
        bi = k % NB
        kn = k + NB - 1
        if kn < NCHT:
            ni = kn % NB
            if wcopy[ni] is not None:
                wcopy[ni].wait()
            gcopy[ni] = start_gather(kn, ni)
        gcopy[bi].wait()

        buf = bufs[bi]
        poff = (k % NCH) * C

        def fma_rows(lo, hi):
            @plsc.parallel_loop(lo, hi, 1)
            def row_body(r):
                @plsc.parallel_loop(0, LGP, 1, unroll=8)
                def col_body(jj):
                    w = pos_v[poff + r, pl.ds(jj * 16, 16)]
                    pa = lax.bitcast_convert_type(w & hi_mask, jnp.float32)
                    pb = lax.bitcast_convert_type(lax.shift_left(w, 16),
                                                  jnp.float32)
                    sa = pl.ds(jj * 32, 16)
                    sb = pl.ds(jj * 32 + 16, 16)
                    buf[r, sa] = buf[r, sa] * SCALE + pa
                    buf[r, sb] = buf[r, sb] * SCALE + pb

        fma_rows(0, C)
        wcopy[bi] = start_write(k, bi)

    for bi in range(NB):
        if wcopy[bi] is not None:
            wcopy[bi].wait()


def kernel(x, table):
    pos = jnp.asarray(_pack_pos_bf16_np(_positional_encoding_np(SEQ, D_MODEL)))
    idx = (
        x.astype(jnp.int32)
        .reshape(BATCH, NW, NCH * C)
        .transpose(1, 0, 2)
        .reshape(NW, BATCH, NCH, C)
    )
    out = _embed(table, idx, pos)
    return out.reshape(BATCH, SEQ, D_MODEL)


# speedup vs baseline: 1.0315x; 1.0315x over previous
import functools

import numpy as np
import jax
import jax.numpy as jnp
from jax import lax
from jax.experimental import pallas as pl
from jax.experimental.pallas import tpu as pltpu
from jax.experimental.pallas import tpu_sc as plsc

VOCAB = 100000
D_MODEL = 768
BATCH = 4
SEQ = 2048
SCALE = float(np.sqrt(float(D_MODEL)))

NC = 2
NS = 16
NW = NC * NS
PB = SEQ // NW
C = 64
NCH = PB // C
LG = D_MODEL // 16
LGP = D_MODEL // 32
NB = 2
NCHT = BATCH * NCH


def _positional_encoding_np(length, depth):
    half_depth = depth // 2
    positions = np.arange(length)[:, np.newaxis]
    exponents = np.arange(half_depth)[np.newaxis, :] * 2 / depth
    denom = 10000 ** exponents
    angles = positions / denom
    pos_encoding = np.zeros((length, depth), dtype=np.float64)
    pos_encoding[:, ::2] = np.sin(angles)
    pos_encoding[:, 1::2] = np.cos(angles)
    return pos_encoding.astype(np.float32)


def _pack_pos_bf16_np(pos):
    u = pos.view(np.uint32).astype(np.uint64)
    bf = ((u + 0x7FFF + ((u >> 16) & 1)) >> 16).astype(np.uint32)
    bf = bf.reshape(pos.shape[0], LGP, 2, 16)
    packed = (bf[:, :, 0, :] << 16) | bf[:, :, 1, :]
    return packed.reshape(pos.shape[0], D_MODEL // 2).view(np.int32)


_MESH = plsc.VectorSubcoreMesh(core_axis_name="c", subcore_axis_name="s")


@functools.partial(
    pl.kernel,
    mesh=_MESH,
    out_type=jax.ShapeDtypeStruct((BATCH * SEQ, D_MODEL), jnp.float32),
    scratch_types=[
        pltpu.VMEM((BATCH, NCH, C), jnp.int32),
        pltpu.VMEM((PB, D_MODEL // 2), jnp.int32),
    ] + [pltpu.VMEM((C, D_MODEL), jnp.float32) for _ in range(NB)]
      + [pltpu.SemaphoreType.DMA for _ in range(2 * NB)],
)
def _embed(table_hbm, idx_hbm, pos_hbm, out_hbm, idx_v, pos_v, *rest):
    bufs = rest[:NB]
    gsems = rest[NB:2 * NB]
    wsems = rest[2 * NB:]

    wid = lax.axis_index("s") * NC + lax.axis_index("c")
    pbase = wid * PB

    pltpu.sync_copy(idx_hbm.at[wid], idx_v)
    pltpu.sync_copy(pos_hbm.at[pl.ds(pbase, PB)], pos_v)

    def start_gather(k, bi):
        b, c = divmod(k, NCH)
        return pltpu.async_copy(table_hbm.at[idx_v.at[b, c]], bufs[bi],
                                gsems[bi])

    def start_write(k, bi):
        b, c = divmod(k, NCH)
        out_base = b * SEQ + pbase + c * C
        return pltpu.async_copy(bufs[bi], out_hbm.at[pl.ds(out_base, C)],
                                wsems[bi])

    gcopy = [None] * NB
    wcopy = [None] * NB
    for k in range(min(NB - 1, NCHT)):
        gcopy[k % NB] = start_gather(k, k % NB)

    hi_mask = jnp.int32(-65536)

    for k in range(NCHT):
        bi = k % NB
        kn = k + NB - 1
        if kn < NCHT:
            ni = kn % NB
            if wcopy[ni] is not None:
                wcopy[ni].wait()
            gcopy[ni] = start_gather(kn, ni)
        gcopy[bi].wait()

        buf = bufs[bi]
        poff = (k % NCH) * C

        def fma_rows(lo, hi):
            @plsc.parallel_loop(lo, hi, 1)
            def row_body(r):
                @plsc.parallel_loop(0, LGP, 1, unroll=4)
                def col_body(jj):
                    w = pos_v[poff + r, pl.ds(jj * 16, 16)]
                    pa = lax.bitcast_convert_type(w & hi_mask, jnp.float32)
                    pb = lax.bitcast_convert_type(lax.shift_left(w, 16),
                                                  jnp.float32)
                    sa = pl.ds(jj * 32, 16)
                    sb = pl.ds(jj * 32 + 16, 16)
                    buf[r, sa] = buf[r, sa] * SCALE + pa
                    buf[r, sb] = buf[r, sb] * SCALE + pb

        if k == NCHT - 1:
            b, c = divmod(k, NCH)
            out_base = b * SEQ + pbase + c * C
            half = C // 2
            fma_rows(0, half)
            t0 = pltpu.async_copy(buf.at[pl.ds(0, half)],
                                  out_hbm.at[pl.ds(out_base, half)],
                                  wsems[bi])
            fma_rows(half, C)
            t1 = pltpu.async_copy(buf.at[pl.ds(half, half)],
                                  out_hbm.at[pl.ds(out_base + half, half)],
                                  wsems[bi])
            t0.wait()
            t1.wait()
        else:
            fma_rows(0, C)
            wcopy[bi] = start_write(k, bi)

    wcopy[(NCHT - 2) % NB].wait()


def kernel(x, table):
    pos = jnp.asarray(_pack_pos_bf16_np(_positional_encoding_np(SEQ, D_MODEL)))
    idx = (
        x.astype(jnp.int32)
        .reshape(BATCH, NW, NCH * C)
        .transpose(1, 0, 2)
        .reshape(NW, BATCH, NCH, C)
    )
    out = _embed(table, idx, pos)
    return out.reshape(BATCH, SEQ, D_MODEL)


# speedup vs baseline: 1.0387x; 1.0069x over previous
import functools

import numpy as np
import jax
import jax.numpy as jnp
from jax import lax
from jax.experimental import pallas as pl
from jax.experimental.pallas import tpu as pltpu
from jax.experimental.pallas import tpu_sc as plsc

VOCAB = 100000
D_MODEL = 768
BATCH = 4
SEQ = 2048
SCALE = float(np.sqrt(float(D_MODEL)))

NC = 2
NS = 16
NW = NC * NS
PB = SEQ // NW
C = 64
NCH = PB // C
LG = D_MODEL // 16
LGP = D_MODEL // 32
NB = 2
NCHT = BATCH * NCH


def _positional_encoding_np(length, depth):
    half_depth = depth // 2
    positions = np.arange(length)[:, np.newaxis]
    exponents = np.arange(half_depth)[np.newaxis, :] * 2 / depth
    denom = 10000 ** exponents
    angles = positions / denom
    pos_encoding = np.zeros((length, depth), dtype=np.float64)
    pos_encoding[:, ::2] = np.sin(angles)
    pos_encoding[:, 1::2] = np.cos(angles)
    return pos_encoding.astype(np.float32)


def _pack_pos_bf16_np(pos):
    u = pos.view(np.uint32).astype(np.uint64)
    bf = ((u + 0x7FFF + ((u >> 16) & 1)) >> 16).astype(np.uint32)
    bf = bf.reshape(pos.shape[0], LGP, 2, 16)
    packed = (bf[:, :, 0, :] << 16) | bf[:, :, 1, :]
    return packed.reshape(pos.shape[0], D_MODEL // 2).view(np.int32)


_MESH = plsc.VectorSubcoreMesh(core_axis_name="c", subcore_axis_name="s")


@functools.partial(
    pl.kernel,
    mesh=_MESH,
    out_type=jax.ShapeDtypeStruct((BATCH * SEQ, D_MODEL), jnp.float32),
    scratch_types=[
        pltpu.VMEM((BATCH, NCH, C), jnp.int32),
        pltpu.VMEM((PB, D_MODEL // 2), jnp.int32),
    ] + [pltpu.VMEM((C, D_MODEL), jnp.float32) for _ in range(NB)]
      + [pltpu.SemaphoreType.DMA for _ in range(2 * NB)],
)
def _embed(table_hbm, idx_hbm, pos_hbm, out_hbm, idx_v, pos_v, *rest):
    bufs = rest[:NB]
    gsems = rest[NB:2 * NB]
    wsems = rest[2 * NB:]

    wid = lax.axis_index("s") * NC + lax.axis_index("c")
    pbase = wid * PB

    pltpu.sync_copy(idx_hbm.at[wid], idx_v)
    pltpu.sync_copy(pos_hbm.at[pl.ds(pbase, PB)], pos_v)

    def start_gather(k, bi):
        b, c = divmod(k, NCH)
        return pltpu.async_copy(table_hbm.at[idx_v.at[b, c]], bufs[bi],
                                gsems[bi])

    def start_write(k, bi):
        b, c = divmod(k, NCH)
        out_base = b * SEQ + pbase + c * C
        return pltpu.async_copy(bufs[bi], out_hbm.at[pl.ds(out_base, C)],
                                wsems[bi])

    gcopy = [None] * NB
    wcopy = [None] * NB
    for k in range(min(NB - 1, NCHT)):
        gcopy[k % NB] = start_gather(k, k % NB)

    hi_mask = jnp.int32(-65536)

    for k in range(NCHT):
        bi = k % NB
        kn = k + NB - 1
        if kn < NCHT:
            ni = kn % NB
            if wcopy[ni] is not None:
                for t in wcopy[ni]:
                    t.wait()
            gcopy[ni] = start_gather(kn, ni)
        gcopy[bi].wait()

        buf = bufs[bi]
        poff = (k % NCH) * C

        def fma_rows(lo, hi):
            @plsc.parallel_loop(lo, hi, 1)
            def row_body(r):
                @plsc.parallel_loop(0, LGP, 1, unroll=4)
                def col_body(jj):
                    w = pos_v[poff + r, pl.ds(jj * 16, 16)]
                    pa = lax.bitcast_convert_type(w & hi_mask, jnp.float32)
                    pb = lax.bitcast_convert_type(lax.shift_left(w, 16),
                                                  jnp.float32)
                    sa = pl.ds(jj * 32, 16)
                    sb = pl.ds(jj * 32 + 16, 16)
                    buf[r, sa] = buf[r, sa] * SCALE + pa
                    buf[r, sb] = buf[r, sb] * SCALE + pb

        b, c = divmod(k, NCH)
        out_base = b * SEQ + pbase + c * C
        half = C // 2
        fma_rows(0, half)
        t0 = pltpu.async_copy(buf.at[pl.ds(0, half)],
                              out_hbm.at[pl.ds(out_base, half)],
                              wsems[bi])
        fma_rows(half, C)
        t1 = pltpu.async_copy(buf.at[pl.ds(half, half)],
                              out_hbm.at[pl.ds(out_base + half, half)],
                              wsems[bi])
        wcopy[bi] = (t0, t1)

    for s in range(NB):
        if wcopy[s] is not None:
            for t in wcopy[s]:
                t.wait()


def kernel(x, table):
    pos = jnp.asarray(_pack_pos_bf16_np(_positional_encoding_np(SEQ, D_MODEL)))
    idx = (
        x.astype(jnp.int32)
        .reshape(BATCH, NW, NCH * C)
        .transpose(1, 0, 2)
        .reshape(NW, BATCH, NCH, C)
    )
    out = _embed(table, idx, pos)
    return out.reshape(BATCH, SEQ, D_MODEL)


# speedup vs baseline: 1.0489x; 1.0098x over previous
import functools

import numpy as np
import jax
import jax.numpy as jnp
from jax import lax
from jax.experimental import pallas as pl
from jax.experimental.pallas import tpu as pltpu
from jax.experimental.pallas import tpu_sc as plsc

VOCAB = 100000
D_MODEL = 768
BATCH = 4
SEQ = 2048
SCALE = float(np.sqrt(float(D_MODEL)))

NC = 2
NS = 16
NW = NC * NS
PB = SEQ // NW
C = 64
NCH = PB // C
LG = D_MODEL // 16
LGP = D_MODEL // 32
NB = 2
NCHT = BATCH * NCH


def _positional_encoding_np(length, depth):
    half_depth = depth // 2
    positions = np.arange(length)[:, np.newaxis]
    exponents = np.arange(half_depth)[np.newaxis, :] * 2 / depth
    denom = 10000 ** exponents
    angles = positions / denom
    pos_encoding = np.zeros((length, depth), dtype=np.float64)
    pos_encoding[:, ::2] = np.sin(angles)
    pos_encoding[:, 1::2] = np.cos(angles)
    return pos_encoding.astype(np.float32)


def _pack_pos_bf16_np(pos):
    u = pos.view(np.uint32).astype(np.uint64)
    bf = ((u + 0x7FFF + ((u >> 16) & 1)) >> 16).astype(np.uint32)
    bf = bf.reshape(pos.shape[0], LGP, 2, 16)
    packed = (bf[:, :, 0, :] << 16) | bf[:, :, 1, :]
    return packed.reshape(pos.shape[0], D_MODEL // 2).view(np.int32)


_MESH = plsc.VectorSubcoreMesh(core_axis_name="c", subcore_axis_name="s")


@functools.partial(
    pl.kernel,
    mesh=_MESH,
    out_type=jax.ShapeDtypeStruct((BATCH * SEQ, D_MODEL), jnp.float32),
    scratch_types=[
        pltpu.VMEM((BATCH, NCH, C), jnp.int32),
        pltpu.VMEM((PB, D_MODEL // 2), jnp.int32),
    ] + [pltpu.VMEM((C, D_MODEL), jnp.float32) for _ in range(NB)]
      + [pltpu.SemaphoreType.DMA for _ in range(2 * NB + 1)],
)
def _embed(table_hbm, idx_hbm, pos_hbm, out_hbm, idx_v, pos_v, *rest):
    bufs = rest[:NB]
    gsems = rest[NB:2 * NB]
    wsems = rest[2 * NB:3 * NB]
    gsem_x = rest[3 * NB]

    wid = lax.axis_index("s") * NC + lax.axis_index("c")
    pbase = wid * PB

    pltpu.sync_copy(idx_hbm.at[wid], idx_v)
    pltpu.sync_copy(pos_hbm.at[pl.ds(pbase, PB)], pos_v)

    def start_gather(k, bi):
        b, c = divmod(k, NCH)
        return pltpu.async_copy(table_hbm.at[idx_v.at[b, c]], bufs[bi],
                                gsems[bi])

    def start_write(k, bi):
        b, c = divmod(k, NCH)
        out_base = b * SEQ + pbase + c * C
        return pltpu.async_copy(bufs[bi], out_hbm.at[pl.ds(out_base, C)],
                                wsems[bi])

    gcopy = [None] * NB
    wcopy = [None] * NB
    halfc = C // 2
    g0a = pltpu.async_copy(table_hbm.at[idx_v.at[0, 0, pl.ds(0, halfc)]],
                           bufs[0].at[pl.ds(0, halfc)], gsems[0])
    g0b = pltpu.async_copy(table_hbm.at[idx_v.at[0, 0, pl.ds(halfc, halfc)]],
                           bufs[0].at[pl.ds(halfc, halfc)], gsem_x)
    gcopy[1 % NB] = start_gather(1, 1 % NB)

    hi_mask = jnp.int32(-65536)

    for k in range(NCHT):
        bi = k % NB
        kn = k + NB - 1
        if k > 0 and kn < NCHT:
            ni = kn % NB
            if wcopy[ni] is not None:
                for t in wcopy[ni]:
                    t.wait()
            gcopy[ni] = start_gather(kn, ni)
        if k == 0:
            g0a.wait()
        else:
            gcopy[bi].wait()

        buf = bufs[bi]
        poff = (k % NCH) * C

        def fma_rows(lo, hi):
            @plsc.parallel_loop(lo, hi, 1)
            def row_body(r):
                @plsc.parallel_loop(0, LGP, 1, unroll=4)
                def col_body(jj):
                    w = pos_v[poff + r, pl.ds(jj * 16, 16)]
                    pa = lax.bitcast_convert_type(w & hi_mask, jnp.float32)
                    pb = lax.bitcast_convert_type(lax.shift_left(w, 16),
                                                  jnp.float32)
                    sa = pl.ds(jj * 32, 16)
                    sb = pl.ds(jj * 32 + 16, 16)
                    buf[r, sa] = buf[r, sa] * SCALE + pa
                    buf[r, sb] = buf[r, sb] * SCALE + pb

        b, c = divmod(k, NCH)
        out_base = b * SEQ + pbase + c * C
        half = C // 2
        fma_rows(0, half)
        t0 = pltpu.async_copy(buf.at[pl.ds(0, half)],
                              out_hbm.at[pl.ds(out_base, half)],
                              wsems[bi])
        if k == 0:
            g0b.wait()
        fma_rows(half, C)
        t1 = pltpu.async_copy(buf.at[pl.ds(half, half)],
                              out_hbm.at[pl.ds(out_base + half, half)],
                              wsems[bi])
        wcopy[bi] = (t0, t1)

    for s in range(NB):
        if wcopy[s] is not None:
            for t in wcopy[s]:
                t.wait()


def kernel(x, table):
    pos = jnp.asarray(_pack_pos_bf16_np(_positional_encoding_np(SEQ, D_MODEL)))
    idx = (
        x.astype(jnp.int32)
        .reshape(BATCH, NW, NCH * C)
        .transpose(1, 0, 2)
        .reshape(NW, BATCH, NCH, C)
    )
    out = _embed(table, idx, pos)
    return out.reshape(BATCH, SEQ, D_MODEL)
